# baseline (speedup 1.0000x reference)
import jax
import jax.numpy as jnp
from jax.experimental import pallas as pl
from jax.experimental.pallas import tpu as pltpu

N_CLASSES = 100000
ROWS = 256
RB = 128
NROWBLK = ROWS // RB
CHUNK = 12544
NCHUNK = 8


def _lse_gather_kernel(x_ref, logits_ref, out_ref, m_ref, s_ref, g_ref):
    c = pl.program_id(1)

    @pl.when(c == 0)
    def _init():
        m_ref[...] = jnp.full((RB, 1), -jnp.inf, jnp.float32)
        s_ref[...] = jnp.zeros((RB, 1), jnp.float32)
        g_ref[...] = jnp.zeros((RB, 1), jnp.float32)

    v = logits_ref[...]
    col = c * CHUNK + jax.lax.broadcasted_iota(jnp.int32, (RB, CHUNK), 1)
    m_old = m_ref[...]

    @pl.when(c < NCHUNK - 1)
    def _full():
        m_new = jnp.maximum(m_old, jnp.max(v, axis=1, keepdims=True))
        s_ref[...] = s_ref[...] * jnp.exp(m_old - m_new) + jnp.sum(
            jnp.exp(v - m_new), axis=1, keepdims=True
        )
        m_ref[...] = m_new
        g_ref[...] += jnp.sum(
            jnp.where(col == x_ref[...], v, 0.0), axis=1, keepdims=True
        )

    @pl.when(c == NCHUNK - 1)
    def _last():
        vm = jnp.where(col < N_CLASSES, v, -jnp.inf)
        m_new = jnp.maximum(m_old, jnp.max(vm, axis=1, keepdims=True))
        s_new = s_ref[...] * jnp.exp(m_old - m_new) + jnp.sum(
            jnp.exp(vm - m_new), axis=1, keepdims=True
        )
        g_new = g_ref[...] + jnp.sum(
            jnp.where(col == x_ref[...], v, 0.0), axis=1, keepdims=True
        )
        out_ref[...] = g_new - m_new - jnp.log(s_new)


def _run(x2, logits2, interpret=False):
    return pl.pallas_call(
        _lse_gather_kernel,
        grid=(NROWBLK, NCHUNK),
        in_specs=[
            pl.BlockSpec((RB, 1), lambda r, c: (r, 0)),
            pl.BlockSpec((RB, CHUNK), lambda r, c: (r, c)),
        ],
        out_specs=pl.BlockSpec((RB, 1), lambda r, c: (r, 0)),
        out_shape=jax.ShapeDtypeStruct((ROWS, 1), jnp.float32),
        scratch_shapes=[
            pltpu.VMEM((RB, 1), jnp.float32),
            pltpu.VMEM((RB, 1), jnp.float32),
            pltpu.VMEM((RB, 1), jnp.float32),
        ],
        compiler_params=pltpu.CompilerParams(
            dimension_semantics=("parallel", "arbitrary"),
        ),
        interpret=interpret,
    )(x2, logits2)


def kernel(x, logits):
    logits2 = logits.reshape(ROWS, N_CLASSES)
    x2 = x.reshape(ROWS, 1).astype(jnp.int32)
    out = _run(x2, logits2)
    return out.reshape(x.shape)


# speedup vs baseline: 1.0586x; 1.0586x over previous
import jax
import jax.numpy as jnp
from jax.experimental import pallas as pl
from jax.experimental.pallas import tpu as pltpu

N_CLASSES = 100000
ROWS = 256
RB = 256
NROWBLK = ROWS // RB
CHUNK = 12544
NCHUNK = 8


def _lse_gather_kernel(x_ref, logits_ref, out_ref, m_ref, s_ref, g_ref):
    c = pl.program_id(1)

    @pl.when(c == 0)
    def _init():
        m_ref[...] = jnp.full((RB, 1), -jnp.inf, jnp.float32)
        s_ref[...] = jnp.zeros((RB, 1), jnp.float32)
        g_ref[...] = jnp.zeros((RB, 1), jnp.float32)

    v = logits_ref[...]
    col = c * CHUNK + jax.lax.broadcasted_iota(jnp.int32, (RB, CHUNK), 1)
    m_old = m_ref[...]

    @pl.when(c < NCHUNK - 1)
    def _full():
        m_new = jnp.maximum(m_old, jnp.max(v, axis=1, keepdims=True))
        s_ref[...] = s_ref[...] * jnp.exp(m_old - m_new) + jnp.sum(
            jnp.exp(v - m_new), axis=1, keepdims=True
        )
        m_ref[...] = m_new
        g_ref[...] += jnp.sum(
            jnp.where(col == x_ref[...], v, 0.0), axis=1, keepdims=True
        )

    @pl.when(c == NCHUNK - 1)
    def _last():
        vm = jnp.where(col < N_CLASSES, v, -jnp.inf)
        m_new = jnp.maximum(m_old, jnp.max(vm, axis=1, keepdims=True))
        s_new = s_ref[...] * jnp.exp(m_old - m_new) + jnp.sum(
            jnp.exp(vm - m_new), axis=1, keepdims=True
        )
        g_new = g_ref[...] + jnp.sum(
            jnp.where(col == x_ref[...], v, 0.0), axis=1, keepdims=True
        )
        out_ref[...] = g_new - m_new - jnp.log(s_new)


def _run(x2, logits2, interpret=False):
    return pl.pallas_call(
        _lse_gather_kernel,
        grid=(NROWBLK, NCHUNK),
        in_specs=[
            pl.BlockSpec((RB, 1), lambda r, c: (r, 0)),
            pl.BlockSpec((RB, CHUNK), lambda r, c: (r, c)),
        ],
        out_specs=pl.BlockSpec((RB, 1), lambda r, c: (r, 0)),
        out_shape=jax.ShapeDtypeStruct((ROWS, 1), jnp.float32),
        scratch_shapes=[
            pltpu.VMEM((RB, 1), jnp.float32),
            pltpu.VMEM((RB, 1), jnp.float32),
            pltpu.VMEM((RB, 1), jnp.float32),
        ],
        compiler_params=pltpu.CompilerParams(
            dimension_semantics=("parallel", "arbitrary"),
        ),
        interpret=interpret,
    )(x2, logits2)


def kernel(x, logits):
    logits2 = logits.reshape(ROWS, N_CLASSES)
    x2 = x.reshape(ROWS, 1).astype(jnp.int32)
    out = _run(x2, logits2)
    return out.reshape(x.shape)
